# Initial kernel scaffold; baseline (speedup 1.0000x reference)
import jax
import jax.numpy as jnp
from jax.experimental import pallas as pl


def kernel(features, edge_index, W1, b1, W2, b2, W3, b3):
    raise NotImplementedError("write your pallas kernel here")


# speedup vs baseline: 6.8814x; 6.8814x over previous
import functools

import jax
import jax.numpy as jnp
from jax import lax
from jax.experimental import pallas as pl
from jax.experimental.pallas import tpu as pltpu
from jax.experimental.pallas import tpu_sc as plsc

N = 10000
E = 320000
D = 128
NC = 2
NS = 16
NW = NC * NS
CHUNK = 128
K = -(-E // (NW * CHUNK))
E_PAD = NW * K * CHUNK
STRIPE = 632
N_PAD = NS * STRIPE
RBLK = 2000
G = N // RBLK

_f32 = jnp.float32


def _sc_aggregate(t, src_w, dst_w, zeros):
    mesh = plsc.VectorSubcoreMesh(core_axis_name="c", subcore_axis_name="s")

    @functools.partial(
        pl.kernel,
        mesh=mesh,
        out_type=jax.ShapeDtypeStruct((NC, N_PAD, D), _f32),
        scratch_types=[
            pltpu.VMEM((K, CHUNK), jnp.int32),
            pltpu.VMEM((K, CHUNK), jnp.int32),
            pltpu.VMEM((CHUNK, D), _f32),
            pltpu.VMEM_SHARED((N_PAD, D), _f32),
            pltpu.SemaphoreType.DMA,
        ],
    )
    def agg(t_hbm, src_hbm, dst_hbm, z_hbm, out_hbm, src_v, dst_v, rows_v,
            acc, sem):
        c = lax.axis_index("c")
        s = lax.axis_index("s")
        wid = c * NS + s
        pltpu.sync_copy(z_hbm, acc.at[pl.ds(s * STRIPE, STRIPE)])
        pltpu.sync_copy(src_hbm.at[wid], src_v)
        pltpu.sync_copy(dst_hbm.at[wid], dst_v)
        plsc.subcore_barrier()

        def body(j, carry):
            pltpu.async_copy(t_hbm.at[src_v.at[j]], rows_v, sem).wait()
            pltpu.sync_copy(rows_v, acc.at[dst_v.at[j]], add=True)
            return carry

        lax.fori_loop(0, K, body, 0)
        plsc.subcore_barrier()
        pltpu.sync_copy(acc.at[pl.ds(s * STRIPE, STRIPE)],
                        out_hbm.at[c, pl.ds(s * STRIPE, STRIPE)])

    return agg(t, src_w, dst_w, zeros)


def _colsum(x):
    def k(x_ref, o_ref):
        @pl.when(pl.program_id(0) == 0)
        def _():
            o_ref[...] = jnp.zeros_like(o_ref)
        o_ref[...] += jnp.sum(x_ref[...], axis=0, keepdims=True)

    return pl.pallas_call(
        k,
        grid=(G,),
        in_specs=[pl.BlockSpec((RBLK, D), lambda i: (i, 0))],
        out_specs=pl.BlockSpec((1, D), lambda i: (0, 0)),
        out_shape=jax.ShapeDtypeStruct((1, D), _f32),
    )(x)


def _pairnorm_matmul(h, cs, w):
    def k(h_ref, cs_ref, w_ref, o_ref):
        x = h_ref[...]
        rn = jnp.sqrt(1e-6 + jnp.sum(x * x, axis=1, keepdims=True))
        xn = x / rn - cs_ref[...] * (1.0 / N)
        o_ref[...] = jnp.dot(xn, w_ref[...], preferred_element_type=_f32)

    return pl.pallas_call(
        k,
        grid=(G,),
        in_specs=[
            pl.BlockSpec((RBLK, D), lambda i: (i, 0)),
            pl.BlockSpec((1, D), lambda i: (0, 0)),
            pl.BlockSpec((D, D), lambda i: (0, 0)),
        ],
        out_specs=pl.BlockSpec((RBLK, D), lambda i: (i, 0)),
        out_shape=jax.ShapeDtypeStruct((N, D), _f32),
    )(h, cs, w)


def _combine_relu_colsum(p0, p1, t, b):
    def k(p0_ref, p1_ref, t_ref, b_ref, h_ref, cs_ref):
        h = jnp.maximum(p0_ref[...] + p1_ref[...] + t_ref[...] + b_ref[...],
                        0.0)
        h_ref[...] = h
        @pl.when(pl.program_id(0) == 0)
        def _():
            cs_ref[...] = jnp.zeros_like(cs_ref)
        cs_ref[...] += jnp.sum(h, axis=0, keepdims=True)

    return pl.pallas_call(
        k,
        grid=(G,),
        in_specs=[
            pl.BlockSpec((RBLK, D), lambda i: (i, 0)),
            pl.BlockSpec((RBLK, D), lambda i: (i, 0)),
            pl.BlockSpec((RBLK, D), lambda i: (i, 0)),
            pl.BlockSpec((1, D), lambda i: (0, 0)),
        ],
        out_specs=[
            pl.BlockSpec((RBLK, D), lambda i: (i, 0)),
            pl.BlockSpec((1, D), lambda i: (0, 0)),
        ],
        out_shape=[
            jax.ShapeDtypeStruct((N, D), _f32),
            jax.ShapeDtypeStruct((1, D), _f32),
        ],
    )(p0, p1, t, b)


def _matmul(h, w):
    def k(h_ref, w_ref, o_ref):
        o_ref[...] = jnp.dot(h_ref[...], w_ref[...],
                             preferred_element_type=_f32)

    return pl.pallas_call(
        k,
        grid=(G,),
        in_specs=[
            pl.BlockSpec((RBLK, D), lambda i: (i, 0)),
            pl.BlockSpec((D, D), lambda i: (0, 0)),
        ],
        out_specs=pl.BlockSpec((RBLK, D), lambda i: (i, 0)),
        out_shape=jax.ShapeDtypeStruct((N, D), _f32),
    )(h, w)


def _combine_final(p0, p1, t, b):
    def k(p0_ref, p1_ref, t_ref, b_ref, o_ref):
        o_ref[...] = p0_ref[...] + p1_ref[...] + t_ref[...] + b_ref[...]

    return pl.pallas_call(
        k,
        grid=(G,),
        in_specs=[
            pl.BlockSpec((RBLK, D), lambda i: (i, 0)),
            pl.BlockSpec((RBLK, D), lambda i: (i, 0)),
            pl.BlockSpec((RBLK, D), lambda i: (i, 0)),
            pl.BlockSpec((1, D), lambda i: (0, 0)),
        ],
        out_specs=pl.BlockSpec((RBLK, D), lambda i: (i, 0)),
        out_shape=jax.ShapeDtypeStruct((N, D), _f32),
    )(p0, p1, t, b)


def kernel(features, edge_index, W1, b1, W2, b2, W3, b3):
    src = edge_index[0].astype(jnp.int32)
    dst = edge_index[1].astype(jnp.int32)
    pad = E_PAD - E
    src_w = jnp.concatenate([src, jnp.zeros((pad,), jnp.int32)]
                            ).reshape(NW, K, CHUNK)
    dst_w = jnp.concatenate([dst, jnp.full((pad,), N, jnp.int32)]
                            ).reshape(NW, K, CHUNK)
    zeros = jnp.zeros((STRIPE, D), _f32)
    b1r = b1.reshape(1, D)
    b2r = b2.reshape(1, D)
    b3r = b3.reshape(1, D)

    cs = _colsum(features)
    t1 = _pairnorm_matmul(features, cs, W1)
    p = _sc_aggregate(t1, src_w, dst_w, zeros)
    h1, cs1 = _combine_relu_colsum(p[0, :N], p[1, :N], t1, b1r)
    t2 = _pairnorm_matmul(h1, cs1, W2)
    p = _sc_aggregate(t2, src_w, dst_w, zeros)
    h2, _ = _combine_relu_colsum(p[0, :N], p[1, :N], t2, b2r)
    t3 = _matmul(h2, W3)
    p = _sc_aggregate(t3, src_w, dst_w, zeros)
    return _combine_final(p[0, :N], p[1, :N], t3, b3r)
